# baseline (speedup 1.0000x reference)
import functools

import jax
import jax.numpy as jnp
from jax import lax
from jax.experimental import pallas as pl
from jax.experimental.pallas import tpu as pltpu
from jax.experimental.pallas import tpu_sc as plsc

N = 10000
E = 160000
D = 256
DH = 128
NC = 2
NS = 16
NACC = 10112
RPT = NACC // NS
C1 = 40
EPAD1 = NC * NS * C1 * 128
C2 = 88
EPAD2 = NS * C2 * 128

_mesh = plsc.VectorSubcoreMesh(
    core_axis_name="c", subcore_axis_name="s", num_cores=NC, num_subcores=NS)


@functools.partial(
    pl.kernel,
    out_type=(jax.ShapeDtypeStruct((NACC, 128), jnp.float32),
              jax.ShapeDtypeStruct((NACC, 128), jnp.float32)),
    mesh=_mesh,
    scratch_types=[
        pltpu.VMEM((C1, 128), jnp.int32),
        pltpu.VMEM((128, 128), jnp.float32),
        pltpu.VMEM_SHARED((NACC, 128), jnp.float32),
        pltpu.SemaphoreType.DMA,
    ],
)
def _sc_hist(col2d, ones_hbm, zeros16, deg_a, deg_b, idx_v, ones_v, acc, sem):
    c = lax.axis_index("c")
    s = lax.axis_index("s")
    w = s * NC + c
    pltpu.sync_copy(col2d.at[pl.ds(w * C1, C1)], idx_v)
    pltpu.sync_copy(ones_hbm, ones_v)
    pltpu.sync_copy(zeros16.at[pl.ds(s * RPT, RPT)], acc.at[pl.ds(s * RPT, RPT)])
    plsc.subcore_barrier()

    def body(gg, carry):
        for f in range(8):
            pltpu.async_copy(ones_v, acc.at[idx_v.at[gg * 8 + f]], sem, add=True)
        for f in range(8):
            pltpu.make_async_copy(ones_v, acc.at[idx_v.at[0]], sem).wait()
        return carry

    lax.fori_loop(0, C1 // 8, body, 0)
    plsc.subcore_barrier()

    @pl.when(c == 0)
    def _():
        pltpu.sync_copy(acc.at[pl.ds(s * RPT, RPT)], deg_a.at[pl.ds(s * RPT, RPT)])

    @pl.when(c == 1)
    def _():
        pltpu.sync_copy(acc.at[pl.ds(s * RPT, RPT)], deg_b.at[pl.ds(s * RPT, RPT)])


@functools.partial(
    pl.kernel,
    out_type=(jax.ShapeDtypeStruct((NACC, DH), jnp.float32),
              jax.ShapeDtypeStruct((NACC, DH), jnp.float32)),
    mesh=_mesh,
    scratch_types=[
        pltpu.VMEM((48, 128), jnp.int32),
        pltpu.VMEM((48, 128), jnp.int32),
        pltpu.VMEM((128, DH), jnp.float32),
        pltpu.VMEM((128, DH), jnp.float32),
        pltpu.VMEM_SHARED((NACC, DH), jnp.float32),
        pltpu.SemaphoreType.DMA,
        pltpu.SemaphoreType.DMA,
    ],
)
def _sc_scatter(row2d, col2d, h_a, h_b, zeros128, s_a, s_b,
                rowv, colv, buf0, buf1, acc, sem0, sem1):
    c = lax.axis_index("c")
    s = lax.axis_index("s")
    pltpu.sync_copy(zeros128.at[pl.ds(s * RPT, RPT)], acc.at[pl.ds(s * RPT, RPT)])
    plsc.subcore_barrier()

    def _edge_loop(h_tab):
        def fire_gather(kk, buf, sem):
            for i in range(4):
                pltpu.async_copy(
                    h_tab.at[rowv.at[kk, pl.ds(32 * i, 32)]],
                    buf.at[pl.ds(32 * i, 32)], sem)

        def wait_gather(buf, sem):
            pltpu.make_async_copy(h_tab.at[rowv.at[0]], buf, sem).wait()

        def half(h0, g):
            pltpu.sync_copy(row2d.at[pl.ds(s * C2 + h0, g)], rowv.at[pl.ds(0, g)])
            pltpu.sync_copy(col2d.at[pl.ds(s * C2 + h0, g)], colv.at[pl.ds(0, g)])
            fire_gather(0, buf0, sem0)

            def body(j2, carry):
                k = 2 * j2
                fire_gather(k + 1, buf1, sem1)
                wait_gather(buf0, sem0)
                pltpu.sync_copy(buf0, acc.at[colv.at[k]], add=True)

                @pl.when(k + 2 < g)
                def _():
                    fire_gather(k + 2, buf0, sem0)

                wait_gather(buf1, sem1)
                pltpu.sync_copy(buf1, acc.at[colv.at[k + 1]], add=True)
                return carry

            lax.fori_loop(0, g // 2, body, 0)

        half(0, 40)
        half(40, 48)

    @pl.when(c == 0)
    def _():
        _edge_loop(h_a)

    @pl.when(c == 1)
    def _():
        _edge_loop(h_b)

    plsc.subcore_barrier()

    @pl.when(c == 0)
    def _():
        pltpu.sync_copy(acc.at[pl.ds(s * RPT, RPT)], s_a.at[pl.ds(s * RPT, RPT)])

    @pl.when(c == 1)
    def _():
        pltpu.sync_copy(acc.at[pl.ds(s * RPT, RPT)], s_b.at[pl.ds(s * RPT, RPT)])


BN = 1000


def _tc_matmul_body(x_ref, wt_ref, h_ref):
    h_ref[...] = jnp.dot(x_ref[...], wt_ref[...],
                         preferred_element_type=jnp.float32,
                         precision=lax.Precision.HIGHEST)


def _tc_matmul(x, wt):
    grid = (N // BN,)
    return pl.pallas_call(
        _tc_matmul_body,
        grid=grid,
        in_specs=[
            pl.BlockSpec((BN, D), lambda i: (i, 0)),
            pl.BlockSpec((D, D), lambda i: (0, 0)),
        ],
        out_specs=pl.BlockSpec((BN, D), lambda i: (i, 0)),
        out_shape=jax.ShapeDtypeStruct((N, D), jnp.float32),
    )(x, wt)


def _tc_scale_body(h_ref, da_ref, db_ref, ha_ref, hb_ref, dinv_ref):
    deg = da_ref[:, 0:1] + db_ref[:, 0:1] + 1.0
    dinv = lax.rsqrt(deg)
    hp = h_ref[...] * dinv
    ha_ref[...] = hp[:, :DH]
    hb_ref[...] = hp[:, DH:]
    dinv_ref[...] = dinv


def _tc_scale(h, deg_a, deg_b):
    grid = (N // BN,)
    return pl.pallas_call(
        _tc_scale_body,
        grid=grid,
        in_specs=[
            pl.BlockSpec((BN, D), lambda i: (i, 0)),
            pl.BlockSpec((BN, 128), lambda i: (i, 0)),
            pl.BlockSpec((BN, 128), lambda i: (i, 0)),
        ],
        out_specs=[
            pl.BlockSpec((BN, DH), lambda i: (i, 0)),
            pl.BlockSpec((BN, DH), lambda i: (i, 0)),
            pl.BlockSpec((BN, 1), lambda i: (i, 0)),
        ],
        out_shape=[
            jax.ShapeDtypeStruct((N, DH), jnp.float32),
            jax.ShapeDtypeStruct((N, DH), jnp.float32),
            jax.ShapeDtypeStruct((N, 1), jnp.float32),
        ],
    )(h, deg_a, deg_b)


def _tc_out_body(sa_ref, sb_ref, dinv_ref, b_ref, o_ref):
    s = jnp.concatenate([sa_ref[...], sb_ref[...]], axis=1)
    o_ref[...] = s * dinv_ref[:, 0:1] + b_ref[...]


def _tc_out(s_a, s_b, dinv, bias):
    grid = (N // BN,)
    return pl.pallas_call(
        _tc_out_body,
        grid=grid,
        in_specs=[
            pl.BlockSpec((BN, DH), lambda i: (i, 0)),
            pl.BlockSpec((BN, DH), lambda i: (i, 0)),
            pl.BlockSpec((BN, 1), lambda i: (i, 0)),
            pl.BlockSpec((1, D), lambda i: (0, 0)),
        ],
        out_specs=pl.BlockSpec((BN, D), lambda i: (i, 0)),
        out_shape=jax.ShapeDtypeStruct((N, D), jnp.float32),
    )(s_a, s_b, dinv, bias)


def kernel(x, edge_index, edge_weight, weights, W, b, selected_idx):
    row = edge_index[0].astype(jnp.int32)
    col = edge_index[1].astype(jnp.int32)
    loop = jnp.arange(N, dtype=jnp.int32)

    col1 = jnp.concatenate(
        [col, N + jnp.arange(EPAD1 - E, dtype=jnp.int32) % (NACC - N)]
    ).reshape(EPAD1 // 128, 128)
    npad = EPAD2 - E - N
    rowf = jnp.concatenate(
        [row, loop, jnp.zeros((npad,), jnp.int32)]).reshape(EPAD2 // 128, 128)
    colf = jnp.concatenate(
        [col, loop, N + jnp.arange(npad, dtype=jnp.int32) % (NACC - N)]
    ).reshape(EPAD2 // 128, 128)

    ones16 = jnp.ones((128, 128), jnp.float32)
    zeros16 = jnp.zeros((NACC, 128), jnp.float32)
    zeros128 = jnp.zeros((NACC, DH), jnp.float32)

    h = _tc_matmul(x, W.T)
    deg_a, deg_b = _sc_hist(col1, ones16, zeros16)
    h_a, h_b, dinv = _tc_scale(h, deg_a[:N], deg_b[:N])
    s_a, s_b = _sc_scatter(rowf, colf, h_a, h_b, zeros128)
    return _tc_out(s_a[:N], s_b[:N], dinv, b.reshape(1, D))


# speedup vs baseline: 1.0744x; 1.0744x over previous
import functools

import jax
import jax.numpy as jnp
from jax import lax
from jax.experimental import pallas as pl
from jax.experimental.pallas import tpu as pltpu
from jax.experimental.pallas import tpu_sc as plsc

N = 10000
E = 160000
D = 256
DH = 128
NC = 2
NS = 16
NACC = 10112
RPT = NACC // NS
C1 = 40
EPAD1 = NC * NS * C1 * 128
C2 = 88
EPAD2 = NS * C2 * 128

_mesh = plsc.VectorSubcoreMesh(
    core_axis_name="c", subcore_axis_name="s", num_cores=NC, num_subcores=NS)


@functools.partial(
    pl.kernel,
    out_type=(jax.ShapeDtypeStruct((NACC, 128), jnp.float32),
              jax.ShapeDtypeStruct((NACC, 128), jnp.float32)),
    mesh=_mesh,
    scratch_types=[
        pltpu.VMEM((C1, 128), jnp.int32),
        pltpu.VMEM((128, 128), jnp.float32),
        pltpu.VMEM_SHARED((NACC, 128), jnp.float32),
        pltpu.SemaphoreType.DMA,
    ],
)
def _sc_hist(col2d, ones_hbm, zeros16, deg_a, deg_b, idx_v, ones_v, acc, sem):
    c = lax.axis_index("c")
    s = lax.axis_index("s")
    w = s * NC + c
    pltpu.sync_copy(col2d.at[pl.ds(w * C1, C1)], idx_v)
    pltpu.sync_copy(ones_hbm, ones_v)
    pltpu.sync_copy(zeros16.at[pl.ds(s * RPT, RPT)], acc.at[pl.ds(s * RPT, RPT)])
    plsc.subcore_barrier()

    def body(gg, carry):
        for f in range(8):
            pltpu.async_copy(ones_v, acc.at[idx_v.at[gg * 8 + f]], sem, add=True)
        for f in range(8):
            pltpu.make_async_copy(ones_v, acc.at[idx_v.at[0]], sem).wait()
        return carry

    lax.fori_loop(0, C1 // 8, body, 0)
    plsc.subcore_barrier()

    @pl.when(c == 0)
    def _():
        pltpu.sync_copy(acc.at[pl.ds(s * RPT, RPT)], deg_a.at[pl.ds(s * RPT, RPT)])

    @pl.when(c == 1)
    def _():
        pltpu.sync_copy(acc.at[pl.ds(s * RPT, RPT)], deg_b.at[pl.ds(s * RPT, RPT)])


@functools.partial(
    pl.kernel,
    out_type=(jax.ShapeDtypeStruct((NACC, DH), jnp.float32),
              jax.ShapeDtypeStruct((NACC, DH), jnp.float32)),
    mesh=_mesh,
    scratch_types=[
        pltpu.VMEM((48, 128), jnp.int32),
        pltpu.VMEM((48, 128), jnp.int32),
        pltpu.VMEM((128, DH), jnp.float32),
        pltpu.VMEM((128, DH), jnp.float32),
        pltpu.VMEM_SHARED((NACC, DH), jnp.float32),
        pltpu.SemaphoreType.DMA,
        pltpu.SemaphoreType.DMA,
    ],
)
def _sc_scatter(row2d, col2d, h_a, h_b, zeros128, s_a, s_b,
                rowv, colv, buf0, buf1, acc, sem0, sem1):
    c = lax.axis_index("c")
    s = lax.axis_index("s")
    pltpu.sync_copy(zeros128.at[pl.ds(s * RPT, RPT)], acc.at[pl.ds(s * RPT, RPT)])
    plsc.subcore_barrier()

    def _edge_loop(h_tab):
        def fire_gather(kk, buf, sem):
            for i in range(4):
                pltpu.async_copy(
                    h_tab.at[rowv.at[kk, pl.ds(32 * i, 32)]],
                    buf.at[pl.ds(32 * i, 32)], sem)

        def wait_gather(buf, sem):
            pltpu.make_async_copy(h_tab.at[rowv.at[0]], buf, sem).wait()

        def half(h0, g):
            pltpu.sync_copy(row2d.at[pl.ds(s * C2 + h0, g)], rowv.at[pl.ds(0, g)])
            pltpu.sync_copy(col2d.at[pl.ds(s * C2 + h0, g)], colv.at[pl.ds(0, g)])
            fire_gather(0, buf0, sem0)

            def body(j2, carry):
                k = 2 * j2
                fire_gather(k + 1, buf1, sem1)
                wait_gather(buf0, sem0)
                pltpu.sync_copy(buf0, acc.at[colv.at[k]], add=True)

                @pl.when(k + 2 < g)
                def _():
                    fire_gather(k + 2, buf0, sem0)

                wait_gather(buf1, sem1)
                pltpu.sync_copy(buf1, acc.at[colv.at[k + 1]], add=True)
                return carry

            lax.fori_loop(0, g // 2, body, 0)

        half(0, 40)
        half(40, 48)

    @pl.when(c == 0)
    def _():
        _edge_loop(h_a)

    @pl.when(c == 1)
    def _():
        _edge_loop(h_b)

    plsc.subcore_barrier()

    @pl.when(c == 0)
    def _():
        pltpu.sync_copy(acc.at[pl.ds(s * RPT, RPT)], s_a.at[pl.ds(s * RPT, RPT)])

    @pl.when(c == 1)
    def _():
        pltpu.sync_copy(acc.at[pl.ds(s * RPT, RPT)], s_b.at[pl.ds(s * RPT, RPT)])


BN = 1000


def _tc_prep_body(x_ref, wt_ref, da_ref, db_ref, ha_ref, hb_ref, dinv_ref):
    deg = da_ref[:, 0:1] + db_ref[:, 0:1] + 1.0
    dinv = lax.rsqrt(deg)
    h = jnp.dot(x_ref[...], wt_ref[...],
                preferred_element_type=jnp.float32,
                precision=lax.Precision.HIGHEST)
    hp = h * dinv
    ha_ref[...] = hp[:, :DH]
    hb_ref[...] = hp[:, DH:]
    dinv_ref[...] = dinv


def _tc_prep(x, wt, deg_a, deg_b):
    grid = (N // BN,)
    return pl.pallas_call(
        _tc_prep_body,
        grid=grid,
        in_specs=[
            pl.BlockSpec((BN, D), lambda i: (i, 0)),
            pl.BlockSpec((D, D), lambda i: (0, 0)),
            pl.BlockSpec((BN, 128), lambda i: (i, 0)),
            pl.BlockSpec((BN, 128), lambda i: (i, 0)),
        ],
        out_specs=[
            pl.BlockSpec((BN, DH), lambda i: (i, 0)),
            pl.BlockSpec((BN, DH), lambda i: (i, 0)),
            pl.BlockSpec((BN, 1), lambda i: (i, 0)),
        ],
        out_shape=[
            jax.ShapeDtypeStruct((N, DH), jnp.float32),
            jax.ShapeDtypeStruct((N, DH), jnp.float32),
            jax.ShapeDtypeStruct((N, 1), jnp.float32),
        ],
    )(x, wt, deg_a, deg_b)


def _tc_out_body(sa_ref, sb_ref, dinv_ref, b_ref, o_ref):
    s = jnp.concatenate([sa_ref[...], sb_ref[...]], axis=1)
    o_ref[...] = s * dinv_ref[:, 0:1] + b_ref[...]


def _tc_out(s_a, s_b, dinv, bias):
    grid = (N // BN,)
    return pl.pallas_call(
        _tc_out_body,
        grid=grid,
        in_specs=[
            pl.BlockSpec((BN, DH), lambda i: (i, 0)),
            pl.BlockSpec((BN, DH), lambda i: (i, 0)),
            pl.BlockSpec((BN, 1), lambda i: (i, 0)),
            pl.BlockSpec((1, D), lambda i: (0, 0)),
        ],
        out_specs=pl.BlockSpec((BN, D), lambda i: (i, 0)),
        out_shape=jax.ShapeDtypeStruct((N, D), jnp.float32),
    )(s_a, s_b, dinv, bias)


def kernel(x, edge_index, edge_weight, weights, W, b, selected_idx):
    row = edge_index[0].astype(jnp.int32)
    col = edge_index[1].astype(jnp.int32)
    loop = jnp.arange(N, dtype=jnp.int32)

    col1 = jnp.concatenate(
        [col, jnp.full((EPAD1 - E,), N, jnp.int32)]).reshape(EPAD1 // 128, 128)
    rowf = jnp.concatenate(
        [row, loop, jnp.zeros((EPAD2 - E - N,), jnp.int32)]).reshape(EPAD2 // 128, 128)
    colf = jnp.concatenate(
        [col, loop, jnp.full((EPAD2 - E - N,), N, jnp.int32)]).reshape(EPAD2 // 128, 128)

    ones16 = jnp.ones((128, 128), jnp.float32)
    zeros16 = jnp.zeros((NACC, 128), jnp.float32)
    zeros128 = jnp.zeros((NACC, DH), jnp.float32)

    deg_a, deg_b = _sc_hist(col1, ones16, zeros16)
    h_a, h_b, dinv = _tc_prep(x, W.T, deg_a[:N], deg_b[:N])
    s_a, s_b = _sc_scatter(rowf, colf, h_a, h_b, zeros128)
    return _tc_out(s_a[:N], s_b[:N], dinv, b.reshape(1, D))


# speedup vs baseline: 1.7860x; 1.6623x over previous
import functools

import jax
import jax.numpy as jnp
from jax import lax
from jax.experimental import pallas as pl
from jax.experimental.pallas import tpu as pltpu
from jax.experimental.pallas import tpu_sc as plsc

N = 10000
E = 160000
D = 256
DH = 128
NC = 2
NS = 16
NACC = 10112
RPT = NACC // NS
C1 = 40
EPAD1 = NC * NS * C1 * 128
C2 = 80
EPAD2 = NS * C2 * 128

_mesh = plsc.VectorSubcoreMesh(
    core_axis_name="c", subcore_axis_name="s", num_cores=NC, num_subcores=NS)


@functools.partial(
    pl.kernel,
    out_type=(jax.ShapeDtypeStruct((NACC, 128), jnp.float32),
              jax.ShapeDtypeStruct((NACC, 128), jnp.float32)),
    mesh=_mesh,
    scratch_types=[
        pltpu.VMEM((C1, 128), jnp.int32),
        pltpu.VMEM((128, 128), jnp.float32),
        pltpu.VMEM_SHARED((NACC, 128), jnp.float32),
        pltpu.SemaphoreType.DMA,
    ],
)
def _sc_hist(col2d, ones_hbm, zeros16, deg_a, deg_b, idx_v, ones_v, acc, sem):
    c = lax.axis_index("c")
    s = lax.axis_index("s")
    w = s * NC + c
    pltpu.sync_copy(col2d.at[pl.ds(w * C1, C1)], idx_v)
    pltpu.sync_copy(ones_hbm, ones_v)
    pltpu.sync_copy(zeros16.at[pl.ds(s * RPT, RPT)], acc.at[pl.ds(s * RPT, RPT)])
    plsc.subcore_barrier()

    def body(gg, carry):
        for f in range(8):
            pltpu.async_copy(ones_v, acc.at[idx_v.at[gg * 8 + f]], sem, add=True)
        for f in range(8):
            pltpu.make_async_copy(ones_v, acc.at[idx_v.at[0]], sem).wait()
        return carry

    lax.fori_loop(0, C1 // 8, body, 0)
    plsc.subcore_barrier()

    @pl.when(c == 0)
    def _():
        pltpu.sync_copy(acc.at[pl.ds(s * RPT, RPT)], deg_a.at[pl.ds(s * RPT, RPT)])

    @pl.when(c == 1)
    def _():
        pltpu.sync_copy(acc.at[pl.ds(s * RPT, RPT)], deg_b.at[pl.ds(s * RPT, RPT)])


@functools.partial(
    pl.kernel,
    out_type=(jax.ShapeDtypeStruct((NACC, DH), jnp.float32),
              jax.ShapeDtypeStruct((NACC, DH), jnp.float32)),
    mesh=_mesh,
    scratch_types=[
        pltpu.VMEM((48, 128), jnp.int32),
        pltpu.VMEM((48, 128), jnp.int32),
        pltpu.VMEM((128, DH), jnp.float32),
        pltpu.VMEM((128, DH), jnp.float32),
        pltpu.VMEM_SHARED((NACC, DH), jnp.float32),
        pltpu.SemaphoreType.DMA,
        pltpu.SemaphoreType.DMA,
    ],
)
def _sc_scatter(row2d, col2d, h_a, h_b, s_a, s_b,
                rowv, colv, buf0, buf1, acc, sem0, sem1):
    c = lax.axis_index("c")
    s = lax.axis_index("s")
    @pl.when(c == 0)
    def _():
        pltpu.sync_copy(h_a.at[pl.ds(s * RPT, RPT)], acc.at[pl.ds(s * RPT, RPT)])

    @pl.when(c == 1)
    def _():
        pltpu.sync_copy(h_b.at[pl.ds(s * RPT, RPT)], acc.at[pl.ds(s * RPT, RPT)])

    plsc.subcore_barrier()

    def _edge_loop(h_tab):
        def fire_gather(kk, buf, sem):
            for i in range(4):
                pltpu.async_copy(
                    h_tab.at[rowv.at[kk, pl.ds(32 * i, 32)]],
                    buf.at[pl.ds(32 * i, 32)], sem)

        def wait_gather(buf, sem):
            pltpu.make_async_copy(h_tab.at[rowv.at[0]], buf, sem).wait()

        def half(h0, g):
            pltpu.sync_copy(row2d.at[pl.ds(s * C2 + h0, g)], rowv.at[pl.ds(0, g)])
            pltpu.sync_copy(col2d.at[pl.ds(s * C2 + h0, g)], colv.at[pl.ds(0, g)])
            fire_gather(0, buf0, sem0)

            def body(j2, carry):
                k = 2 * j2
                fire_gather(k + 1, buf1, sem1)
                wait_gather(buf0, sem0)
                pltpu.sync_copy(buf0, acc.at[colv.at[k]], add=True)

                @pl.when(k + 2 < g)
                def _():
                    fire_gather(k + 2, buf0, sem0)

                wait_gather(buf1, sem1)
                pltpu.sync_copy(buf1, acc.at[colv.at[k + 1]], add=True)
                return carry

            lax.fori_loop(0, g // 2, body, 0)

        half(0, 40)
        half(40, 40)

    @pl.when(c == 0)
    def _():
        _edge_loop(h_a)

    @pl.when(c == 1)
    def _():
        _edge_loop(h_b)

    plsc.subcore_barrier()

    @pl.when(c == 0)
    def _():
        pltpu.sync_copy(acc.at[pl.ds(s * RPT, RPT)], s_a.at[pl.ds(s * RPT, RPT)])

    @pl.when(c == 1)
    def _():
        pltpu.sync_copy(acc.at[pl.ds(s * RPT, RPT)], s_b.at[pl.ds(s * RPT, RPT)])


BN = 1000


def _tc_prep_body(x_ref, wt_ref, da_ref, db_ref, ha_ref, hb_ref, dinv_ref):
    deg = da_ref[:, 0:1] + db_ref[:, 0:1] + 1.0
    dinv = lax.rsqrt(deg)
    h = jnp.dot(x_ref[...], wt_ref[...],
                preferred_element_type=jnp.float32,
                precision=lax.Precision.HIGHEST)
    hp = h * dinv
    ha_ref[...] = hp[:, :DH]
    hb_ref[...] = hp[:, DH:]
    dinv_ref[...] = dinv


def _tc_prep(x, wt, deg_a, deg_b):
    grid = (N // BN,)
    return pl.pallas_call(
        _tc_prep_body,
        grid=grid,
        in_specs=[
            pl.BlockSpec((BN, D), lambda i: (i, 0)),
            pl.BlockSpec((D, D), lambda i: (0, 0)),
            pl.BlockSpec((BN, 128), lambda i: (i, 0)),
            pl.BlockSpec((BN, 128), lambda i: (i, 0)),
        ],
        out_specs=[
            pl.BlockSpec((BN, DH), lambda i: (i, 0)),
            pl.BlockSpec((BN, DH), lambda i: (i, 0)),
            pl.BlockSpec((BN, 1), lambda i: (i, 0)),
        ],
        out_shape=[
            jax.ShapeDtypeStruct((N, DH), jnp.float32),
            jax.ShapeDtypeStruct((N, DH), jnp.float32),
            jax.ShapeDtypeStruct((N, 1), jnp.float32),
        ],
    )(x, wt, deg_a, deg_b)


def _tc_out_body(sa_ref, sb_ref, dinv_ref, b_ref, o_ref):
    s = jnp.concatenate([sa_ref[...], sb_ref[...]], axis=1)
    o_ref[...] = s * dinv_ref[:, 0:1] + b_ref[...]


def _tc_out(s_a, s_b, dinv, bias):
    grid = (N // BN,)
    return pl.pallas_call(
        _tc_out_body,
        grid=grid,
        in_specs=[
            pl.BlockSpec((BN, DH), lambda i: (i, 0)),
            pl.BlockSpec((BN, DH), lambda i: (i, 0)),
            pl.BlockSpec((BN, 1), lambda i: (i, 0)),
            pl.BlockSpec((1, D), lambda i: (0, 0)),
        ],
        out_specs=pl.BlockSpec((BN, D), lambda i: (i, 0)),
        out_shape=jax.ShapeDtypeStruct((N, D), jnp.float32),
    )(s_a, s_b, dinv, bias)


def kernel(x, edge_index, edge_weight, weights, W, b, selected_idx):
    row = edge_index[0].astype(jnp.int32)
    col = edge_index[1].astype(jnp.int32)

    col1 = jnp.concatenate(
        [col, jnp.full((EPAD1 - E,), N, jnp.int32)]).reshape(EPAD1 // 128, 128)
    rowf = jnp.concatenate(
        [row, jnp.zeros((EPAD2 - E,), jnp.int32)]).reshape(EPAD2 // 128, 128)
    colf = jnp.concatenate(
        [col, jnp.full((EPAD2 - E,), N, jnp.int32)]).reshape(EPAD2 // 128, 128)

    ones16 = jnp.ones((128, 128), jnp.float32)
    zeros16 = jnp.zeros((NACC, 128), jnp.float32)

    deg_a, deg_b = _sc_hist(col1, ones16, zeros16)
    h_a, h_b, dinv = _tc_prep(x, W.T, deg_a[:N], deg_b[:N])
    pad = ((0, NACC - N), (0, 0))
    s_a, s_b = _sc_scatter(rowf, colf, jnp.pad(h_a, pad), jnp.pad(h_b, pad))
    return _tc_out(s_a[:N], s_b[:N], dinv, b.reshape(1, D))


# speedup vs baseline: 1.8110x; 1.0140x over previous
import functools

import jax
import jax.numpy as jnp
from jax import lax
from jax.experimental import pallas as pl
from jax.experimental.pallas import tpu as pltpu
from jax.experimental.pallas import tpu_sc as plsc

N = 10000
E = 160000
D = 256
DH = 128
NC = 2
NS = 16
NACC = 10112
RPT = NACC // NS
C1 = 40
EPAD1 = NC * NS * C1 * 128
C2 = 80
EPAD2 = NS * C2 * 128

_mesh = plsc.VectorSubcoreMesh(
    core_axis_name="c", subcore_axis_name="s", num_cores=NC, num_subcores=NS)


@functools.partial(
    pl.kernel,
    out_type=(jax.ShapeDtypeStruct((NACC, 128), jnp.float32),
              jax.ShapeDtypeStruct((NACC, 128), jnp.float32)),
    mesh=_mesh,
    scratch_types=[
        pltpu.VMEM((C1, 128), jnp.int32),
        pltpu.VMEM((128, 128), jnp.float32),
        pltpu.VMEM_SHARED((NACC, 128), jnp.float32),
        pltpu.SemaphoreType.DMA,
    ],
)
def _sc_hist(col2d, ones_hbm, zeros16, deg_a, deg_b, idx_v, ones_v, acc, sem):
    c = lax.axis_index("c")
    s = lax.axis_index("s")
    w = s * NC + c
    pltpu.sync_copy(col2d.at[pl.ds(w * C1, C1)], idx_v)
    pltpu.sync_copy(ones_hbm, ones_v)
    pltpu.sync_copy(zeros16.at[pl.ds(s * RPT, RPT)], acc.at[pl.ds(s * RPT, RPT)])
    plsc.subcore_barrier()

    def body(gg, carry):
        for f in range(8):
            pltpu.async_copy(ones_v, acc.at[idx_v.at[gg * 8 + f]], sem, add=True)
        for f in range(8):
            pltpu.make_async_copy(ones_v, acc.at[idx_v.at[0]], sem).wait()
        return carry

    lax.fori_loop(0, C1 // 8, body, 0)
    plsc.subcore_barrier()

    @pl.when(c == 0)
    def _():
        pltpu.sync_copy(acc.at[pl.ds(s * RPT, RPT)], deg_a.at[pl.ds(s * RPT, RPT)])

    @pl.when(c == 1)
    def _():
        pltpu.sync_copy(acc.at[pl.ds(s * RPT, RPT)], deg_b.at[pl.ds(s * RPT, RPT)])


@functools.partial(
    pl.kernel,
    out_type=(jax.ShapeDtypeStruct((NACC, DH), jnp.float32),
              jax.ShapeDtypeStruct((NACC, DH), jnp.float32)),
    mesh=_mesh,
    scratch_types=[
        pltpu.VMEM((48, 128), jnp.int32),
        pltpu.VMEM((48, 128), jnp.int32),
        pltpu.VMEM((128, DH), jnp.float32),
        pltpu.VMEM((128, DH), jnp.float32),
        pltpu.VMEM_SHARED((NACC, DH), jnp.float32),
        pltpu.SemaphoreType.DMA,
        pltpu.SemaphoreType.DMA,
    ],
)
def _sc_scatter(row2d, col2d, h_a, h_b, s_a, s_b,
                rowv, colv, buf0, buf1, acc, sem0, sem1):
    c = lax.axis_index("c")
    s = lax.axis_index("s")
    @pl.when(c == 0)
    def _():
        pltpu.sync_copy(h_a.at[pl.ds(s * RPT, RPT)], acc.at[pl.ds(s * RPT, RPT)])

    @pl.when(c == 1)
    def _():
        pltpu.sync_copy(h_b.at[pl.ds(s * RPT, RPT)], acc.at[pl.ds(s * RPT, RPT)])

    plsc.subcore_barrier()

    def _edge_loop(h_tab):
        def fire_gather(kk, buf, sem):
            for i in range(4):
                pltpu.async_copy(
                    h_tab.at[rowv.at[kk, pl.ds(32 * i, 32)]],
                    buf.at[pl.ds(32 * i, 32)], sem)

        def wait_gather(buf, sem):
            pltpu.make_async_copy(h_tab.at[rowv.at[0]], buf, sem).wait()

        def half(h0, g):
            pltpu.sync_copy(row2d.at[pl.ds(s * C2 + h0, g)], rowv.at[pl.ds(0, g)])
            pltpu.sync_copy(col2d.at[pl.ds(s * C2 + h0, g)], colv.at[pl.ds(0, g)])
            fire_gather(0, buf0, sem0)

            def body(j2, carry):
                k = 2 * j2
                fire_gather(k + 1, buf1, sem1)
                wait_gather(buf0, sem0)
                pltpu.sync_copy(buf0, acc.at[colv.at[k]], add=True)

                @pl.when(k + 2 < g)
                def _():
                    fire_gather(k + 2, buf0, sem0)

                wait_gather(buf1, sem1)
                pltpu.sync_copy(buf1, acc.at[colv.at[k + 1]], add=True)
                return carry

            lax.fori_loop(0, g // 2, body, 0)

        half(0, 40)
        half(40, 40)

    @pl.when(c == 0)
    def _():
        _edge_loop(h_a)

    @pl.when(c == 1)
    def _():
        _edge_loop(h_b)

    plsc.subcore_barrier()

    @pl.when(c == 0)
    def _():
        pltpu.sync_copy(acc.at[pl.ds(s * RPT, RPT)], s_a.at[pl.ds(s * RPT, RPT)])

    @pl.when(c == 1)
    def _():
        pltpu.sync_copy(acc.at[pl.ds(s * RPT, RPT)], s_b.at[pl.ds(s * RPT, RPT)])


BN = 1000


def _tc_prep_body(x_ref, wt_ref, da_ref, db_ref, ha_ref, hb_ref, dinv_ref):
    deg = da_ref[:, 0:1] + db_ref[:, 0:1] + 1.0
    dinv = lax.rsqrt(deg)
    h = jnp.dot(x_ref[...], wt_ref[...],
                preferred_element_type=jnp.float32,
                precision=lax.Precision.HIGHEST)
    hp = h * dinv
    ha_ref[...] = hp[:, :DH]
    hb_ref[...] = hp[:, DH:]
    dinv_ref[...] = dinv


def _tc_prep(x, wt, deg_a, deg_b):
    grid = (N // BN,)
    return pl.pallas_call(
        _tc_prep_body,
        grid=grid,
        in_specs=[
            pl.BlockSpec((BN, D), lambda i: (i, 0)),
            pl.BlockSpec((D, D), lambda i: (0, 0)),
            pl.BlockSpec((BN, 128), lambda i: (i, 0)),
            pl.BlockSpec((BN, 128), lambda i: (i, 0)),
        ],
        out_specs=[
            pl.BlockSpec((BN, DH), lambda i: (i, 0)),
            pl.BlockSpec((BN, DH), lambda i: (i, 0)),
            pl.BlockSpec((BN, 1), lambda i: (i, 0)),
        ],
        out_shape=[
            jax.ShapeDtypeStruct((N, DH), jnp.float32),
            jax.ShapeDtypeStruct((N, DH), jnp.float32),
            jax.ShapeDtypeStruct((N, 1), jnp.float32),
        ],
    )(x, wt, deg_a, deg_b)


def _tc_out_body(sa_ref, sb_ref, dinv_ref, b_ref, o_ref):
    s = jnp.concatenate([sa_ref[...], sb_ref[...]], axis=1)
    o_ref[...] = s * dinv_ref[:, 0:1] + b_ref[...]


def _tc_out(s_a, s_b, dinv, bias):
    grid = (N // BN,)
    return pl.pallas_call(
        _tc_out_body,
        grid=grid,
        in_specs=[
            pl.BlockSpec((BN, DH), lambda i: (i, 0)),
            pl.BlockSpec((BN, DH), lambda i: (i, 0)),
            pl.BlockSpec((BN, 1), lambda i: (i, 0)),
            pl.BlockSpec((1, D), lambda i: (0, 0)),
        ],
        out_specs=pl.BlockSpec((BN, D), lambda i: (i, 0)),
        out_shape=jax.ShapeDtypeStruct((N, D), jnp.float32),
    )(s_a, s_b, dinv, bias)


def kernel(x, edge_index, edge_weight, weights, W, b, selected_idx):
    row = edge_index[0].astype(jnp.int32)
    col = edge_index[1].astype(jnp.int32)

    col1 = jnp.concatenate(
        [col, jnp.full((EPAD1 - E,), N, jnp.int32)]).reshape(EPAD1 // 128, 128)
    rowf = jnp.concatenate(
        [row, jnp.zeros((EPAD2 - E,), jnp.int32)]).reshape(EPAD2 // 128, 128)
    colf = jnp.concatenate(
        [col, N + jnp.arange(EPAD2 - E, dtype=jnp.int32) % (NACC - N)]
    ).reshape(EPAD2 // 128, 128)

    ones16 = jnp.ones((128, 128), jnp.float32)
    zeros16 = jnp.zeros((NACC, 128), jnp.float32)

    deg_a, deg_b = _sc_hist(col1, ones16, zeros16)
    h_a, h_b, dinv = _tc_prep(x, W.T, deg_a[:N], deg_b[:N])
    pad = ((0, NACC - N), (0, 0))
    s_a, s_b = _sc_scatter(rowf, colf, jnp.pad(h_a, pad), jnp.pad(h_b, pad))
    return _tc_out(s_a[:N], s_b[:N], dinv, b.reshape(1, D))
